# baseline (speedup 1.0000x reference)
import functools

import jax
import jax.numpy as jnp
from jax import lax
from jax.experimental import pallas as pl
from jax.experimental.pallas import tpu as pltpu
from jax.experimental.pallas import tpu_sc as plsc

NN = 10000
NE = 320000
DF = 128
NH = 64
NCLS = 16

NC = 2
NS = 16
NW = NC * NS
EPW = NE // NW
CHUNK = 100
NITER = EPW // CHUNK
NITER2 = 2 * NITER
NPAD = 10240
RPT = NPAD // NS
LAG = 8

_SC_MESH = plsc.VectorSubcoreMesh(core_axis_name="c", subcore_axis_name="s")
_SC_PARAMS = pltpu.CompilerParams(use_tc_tiling_on_sc=False,
                                  needs_layout_passes=False)


def _mega_kernel(src_hbm, dst_hbm, tab_hbm,
                 acc_hbm, tabs_hbm, normo_hbm, normi_hbm,
                 idxs_all, idxd_all, rows0, rows1, tabbuf, degbuf,
                 normo_v, normi_v, ones_v, zbuf, zdeg,
                 acc_sh, dego_sh, degi_sh,
                 isem, tsem, g0, g1, ssem):
    c = lax.axis_index("c")
    s = lax.axis_index("s")

    cps = pltpu.async_copy(src_hbm.at[s], idxs_all, isem)
    cpd = pltpu.async_copy(dst_hbm.at[s], idxd_all, isem)
    HRP = RPT // 2
    cpt = pltpu.async_copy(tab_hbm.at[pl.ds(s * RPT, HRP)], tabbuf, tsem)

    def fill(i, carry):
        ones_v[pl.ds(i * 16, 16)] = jnp.ones((16,), jnp.float32)
        return carry

    lax.fori_loop(0, 7, fill, 0)

    z16 = jnp.zeros((16,), jnp.float32)

    def zdfill(i, carry):
        zdeg[pl.ds(i * 16, 16)] = z16
        return carry

    lax.fori_loop(0, RPT // 16, zdfill, 0)

    def zero_row(i, carry):
        for jj in range(NH // 16):
            zbuf[i, pl.ds(jj * 16, 16)] = z16
        return carry

    lax.fori_loop(0, 64, zero_row, 0)

    pltpu.sync_copy(zdeg, dego_sh.at[pl.ds(s * RPT, RPT)])
    pltpu.sync_copy(zdeg, degi_sh.at[pl.ds(s * RPT, RPT)])

    def zero_slice(k, carry):
        pltpu.sync_copy(zbuf, acc_sh.at[pl.ds(s * RPT + k * 64, 64)])
        return carry

    lax.fori_loop(0, RPT // 64, zero_slice, 0)

    cps.wait()
    cpd.wait()
    plsc.subcore_barrier()

    ones_c = ones_v.at[pl.ds(0, CHUNK)]

    def hbody(j, carry):
        @pl.when(j >= LAG)
        def _drain():
            pltpu.make_async_copy(ones_c, dego_sh.at[idxs_all.at[0]],
                                  ssem).wait()
            pltpu.make_async_copy(ones_c, degi_sh.at[idxd_all.at[0]],
                                  ssem).wait()

        pltpu.async_copy(ones_c, dego_sh.at[idxs_all.at[j]], ssem, add=True)
        pltpu.async_copy(ones_c, degi_sh.at[idxd_all.at[j]], ssem, add=True)
        return carry

    lax.fori_loop(0, NITER2, hbody, 0)

    def drain(j, carry):
        pltpu.make_async_copy(ones_c, dego_sh.at[idxs_all.at[0]], ssem).wait()
        pltpu.make_async_copy(ones_c, degi_sh.at[idxd_all.at[0]], ssem).wait()
        return carry

    lax.fori_loop(0, LAG, drain, 0)
    plsc.subcore_barrier()

    MAGIC = jnp.int32(0x5F3759DF)

    def _norms(deg_sh, out_v):
        pltpu.sync_copy(deg_sh.at[pl.ds(s * RPT, RPT)], degbuf)

        def nbody(g, carry):
            d = jnp.maximum(degbuf[pl.ds(g * 16, 16)], 1.0)
            ii = MAGIC - (plsc.bitcast(d, jnp.int32) >> 1)
            y = plsc.bitcast(ii, jnp.float32)
            y = y * (1.5 - 0.5 * d * y * y)
            y = y * (1.5 - 0.5 * d * y * y)
            y = y * (1.5 - 0.5 * d * y * y)
            out_v[pl.ds(g * 16, 16)] = y
            return carry

        lax.fori_loop(0, RPT // 16, nbody, 0)

    _norms(dego_sh, normo_v)
    _norms(degi_sh, normi_v)

    @pl.when(c == 0)
    def _write_norms():
        pltpu.sync_copy(normo_v, normo_hbm.at[pl.ds(s * RPT, RPT)])
        pltpu.sync_copy(normi_v, normi_hbm.at[pl.ds(s * RPT, RPT)])

    cpt.wait()

    def scale_half(h):
        def scale_row(r, carry):
            nb = plsc.load_gather(
                normo_v, [jnp.full((16,), h * HRP + r, jnp.int32)])
            for jj in range(NH // 16):
                tabbuf[r, pl.ds(jj * 16, 16)] = (
                    tabbuf[r, pl.ds(jj * 16, 16)] * nb)
            return carry

        lax.fori_loop(0, HRP, scale_row, 0)
        pltpu.sync_copy(tabbuf, tabs_hbm.at[pl.ds(s * RPT + h * HRP, HRP)])

    scale_half(0)
    pltpu.sync_copy(tab_hbm.at[pl.ds(s * RPT + HRP, HRP)], tabbuf)
    scale_half(1)
    plsc.subcore_barrier()

    base = c * NITER
    pltpu.async_copy(tabs_hbm.at[idxs_all.at[base]], rows0, g0)
    pltpu.async_copy(tabs_hbm.at[idxs_all.at[base + 1]], rows1, g1)

    def body(i, carry):
        j0 = base + 2 * i
        j1 = j0 + 1
        pltpu.make_async_copy(tabs_hbm.at[idxs_all.at[j0]], rows0, g0).wait()
        pltpu.sync_copy(rows0, acc_sh.at[idxd_all.at[j0]], add=True)

        @pl.when(2 * i + 2 < NITER)
        def _g0():
            pltpu.async_copy(tabs_hbm.at[idxs_all.at[j0 + 2]], rows0, g0)

        pltpu.make_async_copy(tabs_hbm.at[idxs_all.at[j1]], rows1, g1).wait()
        pltpu.sync_copy(rows1, acc_sh.at[idxd_all.at[j1]], add=True)

        @pl.when(2 * i + 3 < NITER)
        def _g1():
            pltpu.async_copy(tabs_hbm.at[idxs_all.at[j1 + 2]], rows1, g1)

        return carry

    lax.fori_loop(0, NITER // 2, body, 0)
    plsc.subcore_barrier()

    pltpu.sync_copy(acc_sh.at[pl.ds(s * RPT, RPT)],
                    acc_hbm.at[c, pl.ds(s * RPT, RPT)])


_mega_call = pl.kernel(
    _mega_kernel,
    out_type=[jax.ShapeDtypeStruct((NC, NPAD, NH), jnp.float32),
              jax.ShapeDtypeStruct((NPAD, NH), jnp.float32),
              jax.ShapeDtypeStruct((NPAD,), jnp.float32),
              jax.ShapeDtypeStruct((NPAD,), jnp.float32)],
    mesh=_SC_MESH,
    compiler_params=_SC_PARAMS,
    scratch_types=[
        pltpu.VMEM((NITER2, CHUNK), jnp.int32),
        pltpu.VMEM((NITER2, CHUNK), jnp.int32),
        pltpu.VMEM((CHUNK, NH), jnp.float32),
        pltpu.VMEM((CHUNK, NH), jnp.float32),
        pltpu.VMEM((RPT // 2, NH), jnp.float32),
        pltpu.VMEM((RPT,), jnp.float32),
        pltpu.VMEM((RPT,), jnp.float32),
        pltpu.VMEM((RPT,), jnp.float32),
        pltpu.VMEM((112,), jnp.float32),
        pltpu.VMEM((64, NH), jnp.float32),
        pltpu.VMEM((RPT,), jnp.float32),
        pltpu.VMEM_SHARED((NPAD, NH), jnp.float32),
        pltpu.VMEM_SHARED((NPAD,), jnp.float32),
        pltpu.VMEM_SHARED((NPAD,), jnp.float32),
        pltpu.SemaphoreType.DMA,
        pltpu.SemaphoreType.DMA,
        pltpu.SemaphoreType.DMA,
        pltpu.SemaphoreType.DMA,
        pltpu.SemaphoreType.DMA,
    ],
)


def _edge_kernel(F, src_hbm, dst_hbm, tab_hbm, acc_hbm,
                 idxs_all, idxd_all, rows0, rows1, zbuf, acc_sh, tab_sh,
                 isem, g0, g1):
    c = lax.axis_index("c")
    s = lax.axis_index("s")
    wid = c * NS + s

    cps = pltpu.async_copy(src_hbm.at[wid], idxs_all, isem)
    cpd = pltpu.async_copy(dst_hbm.at[wid], idxd_all, isem)
    cpt = pltpu.async_copy(tab_hbm.at[pl.ds(s * RPT, RPT)],
                           tab_sh.at[pl.ds(s * RPT, RPT)], isem)

    zrows = 128
    z16 = jnp.zeros((16,), jnp.float32)

    def zero_row(i, carry):
        for jj in range(F // 16):
            zbuf[i, pl.ds(jj * 16, 16)] = z16
        return carry

    lax.fori_loop(0, zrows, zero_row, 0)

    def zero_slice(k, carry):
        pltpu.sync_copy(zbuf, acc_sh.at[pl.ds(s * RPT + k * zrows, zrows)])
        return carry

    lax.fori_loop(0, RPT // zrows, zero_slice, 0)
    cps.wait()
    cpd.wait()
    cpt.wait()
    plsc.subcore_barrier()

    pltpu.async_copy(tab_sh.at[idxs_all.at[0]], rows0, g0)
    pltpu.async_copy(tab_sh.at[idxs_all.at[1]], rows1, g1)

    def body(i, carry):
        j0 = 2 * i
        j1 = j0 + 1
        pltpu.make_async_copy(tab_sh.at[idxs_all.at[j0]], rows0, g0).wait()
        pltpu.sync_copy(rows0, acc_sh.at[idxd_all.at[j0]], add=True)

        @pl.when(j0 + 2 < NITER)
        def _g0():
            pltpu.async_copy(tab_sh.at[idxs_all.at[j0 + 2]], rows0, g0)

        pltpu.make_async_copy(tab_sh.at[idxs_all.at[j1]], rows1, g1).wait()
        pltpu.sync_copy(rows1, acc_sh.at[idxd_all.at[j1]], add=True)

        @pl.when(j1 + 2 < NITER)
        def _g1():
            pltpu.async_copy(tab_sh.at[idxs_all.at[j1 + 2]], rows1, g1)

        return carry

    lax.fori_loop(0, NITER // 2, body, 0)
    plsc.subcore_barrier()

    pltpu.sync_copy(acc_sh.at[pl.ds(s * RPT, RPT)],
                    acc_hbm.at[c, pl.ds(s * RPT, RPT)])


_edge_call_c = pl.kernel(
    functools.partial(_edge_kernel, NCLS),
    out_type=jax.ShapeDtypeStruct((NC, NPAD, NCLS), jnp.float32),
    mesh=_SC_MESH,
    compiler_params=_SC_PARAMS,
    scratch_types=[
        pltpu.VMEM((NITER, CHUNK), jnp.int32),
        pltpu.VMEM((NITER, CHUNK), jnp.int32),
        pltpu.VMEM((CHUNK, NCLS), jnp.float32),
        pltpu.VMEM((CHUNK, NCLS), jnp.float32),
        pltpu.VMEM((128, NCLS), jnp.float32),
        pltpu.VMEM_SHARED((NPAD, NCLS), jnp.float32),
        pltpu.VMEM_SHARED((NPAD, NCLS), jnp.float32),
        pltpu.SemaphoreType.DMA,
        pltpu.SemaphoreType.DMA,
        pltpu.SemaphoreType.DMA,
    ],
)


def _mm_body(x_ref, w_ref, o_ref):
    z = jnp.dot(x_ref[...], w_ref[...], preferred_element_type=jnp.float32)
    o_ref[0:NN, :] = z
    o_ref[NN:NPAD, :] = jnp.zeros((NPAD - NN, NH), jnp.float32)


_mm = pl.pallas_call(
    _mm_body,
    out_shape=jax.ShapeDtypeStruct((NPAD, NH), jnp.float32),
)


def _mid_body(acc_ref, ni_ref, no_ref, b1_ref, w2_ref, o_ref):
    h = acc_ref[0] + acc_ref[1]
    h = jnp.maximum(h * ni_ref[...] + b1_ref[...], 0.0)
    o_ref[...] = jnp.dot(h * no_ref[...], w2_ref[...],
                         preferred_element_type=jnp.float32)


_mid = pl.pallas_call(
    _mid_body,
    out_shape=jax.ShapeDtypeStruct((NPAD, NCLS), jnp.float32),
)


def _fin_body(acc_ref, ni_ref, b2_ref, o_ref):
    o_ref[...] = ((acc_ref[0, :NN, :] + acc_ref[1, :NN, :])
                  * ni_ref[...] + b2_ref[...])


_fin = pl.pallas_call(
    _fin_body,
    out_shape=jax.ShapeDtypeStruct((NN, NCLS), jnp.float32),
)


@jax.jit
def kernel(features, edge_index, W1, b1, W2, b2):
    src_h = edge_index[0].reshape(NS, NITER2, CHUNK)
    dst_h = edge_index[1].reshape(NS, NITER2, CHUNK)
    src_e = edge_index[0].reshape(NW, NITER, CHUNK)
    dst_e = edge_index[1].reshape(NW, NITER, CHUNK)

    z1 = _mm(features, W1)
    acc1, _tabs, normo, normi = _mega_call(src_h, dst_h, z1)
    ni_c = normi.reshape(NPAD, 1)
    no_c = normo.reshape(NPAD, 1)
    hw2 = _mid(acc1, ni_c, no_c, b1.reshape(1, NH), W2)
    acc2 = _edge_call_c(src_e, dst_e, hw2)
    return _fin(acc2, ni_c[:NN], b2.reshape(1, NCLS))


# speedup vs baseline: 1.1777x; 1.1777x over previous
import functools

import jax
import jax.numpy as jnp
from jax import lax
from jax.experimental import pallas as pl
from jax.experimental.pallas import tpu as pltpu
from jax.experimental.pallas import tpu_sc as plsc

NN = 10000
NE = 320000
DF = 128
NH = 64
NCLS = 16

NC = 2
NS = 16
NW = NC * NS
EPW = NE // NW
CHUNK = 100
NITER = EPW // CHUNK
NPAD = 10240
RPT = NPAD // NS
LAG = 8

_SC_MESH = plsc.VectorSubcoreMesh(core_axis_name="c", subcore_axis_name="s")
_SC_PARAMS = pltpu.CompilerParams(use_tc_tiling_on_sc=False)


def _edge_ring(tab, acc_sh, idxs_all, idxd_all, rows, gsems, ssems):
    for b in range(3):
        pltpu.async_copy(tab.at[idxs_all.at[b]], rows[b], gsems[b])

    def body(i, carry):
        for u in range(4):
            j = 4 * i + u
            pltpu.make_async_copy(tab.at[idxs_all.at[j]], rows[u],
                                  gsems[u]).wait()
            pltpu.async_copy(rows[u], acc_sh.at[idxd_all.at[j]], ssems[u],
                             add=True)

            @pl.when(j >= 1)
            def _ws():
                pltpu.make_async_copy(rows[(u - 1) % 4],
                                      acc_sh.at[idxd_all.at[0]],
                                      ssems[(u - 1) % 4]).wait()

            @pl.when(j + 3 < NITER)
            def _g():
                pltpu.async_copy(tab.at[idxs_all.at[j + 3]],
                                 rows[(u + 3) % 4], gsems[(u + 3) % 4])

        return carry

    lax.fori_loop(0, NITER // 4, body, 0)
    pltpu.make_async_copy(rows[3], acc_sh.at[idxd_all.at[0]],
                          ssems[3]).wait()


def _deg_kernel(src_hbm, dst_hbm, dego_hbm, degi_hbm, idxs_all, idxd_all,
                ones_v, zbuf, dego_sh, degi_sh, isem, ssem):
    c = lax.axis_index("c")
    s = lax.axis_index("s")
    wid = c * NS + s

    cps = pltpu.async_copy(src_hbm.at[wid], idxs_all, isem)
    cpd = pltpu.async_copy(dst_hbm.at[wid], idxd_all, isem)

    def fill(i, carry):
        ones_v[pl.ds(i * 16, 16)] = jnp.ones((16,), jnp.float32)
        return carry

    lax.fori_loop(0, 7, fill, 0)

    def zfill(i, carry):
        zbuf[pl.ds(i * 16, 16)] = jnp.zeros((16,), jnp.float32)
        return carry

    lax.fori_loop(0, RPT // 16, zfill, 0)
    pltpu.sync_copy(zbuf, dego_sh.at[pl.ds(s * RPT, RPT)])
    pltpu.sync_copy(zbuf, degi_sh.at[pl.ds(s * RPT, RPT)])
    cps.wait()
    cpd.wait()
    plsc.subcore_barrier()

    ones_c = ones_v.at[pl.ds(0, CHUNK)]

    def body(j, carry):
        @pl.when(j >= LAG)
        def _drain():
            pltpu.make_async_copy(ones_c, dego_sh.at[idxs_all.at[0]],
                                  ssem).wait()
            pltpu.make_async_copy(ones_c, degi_sh.at[idxd_all.at[0]],
                                  ssem).wait()

        pltpu.async_copy(ones_c, dego_sh.at[idxs_all.at[j]], ssem, add=True)
        pltpu.async_copy(ones_c, degi_sh.at[idxd_all.at[j]], ssem, add=True)
        return carry

    lax.fori_loop(0, NITER, body, 0)

    def drain(j, carry):
        pltpu.make_async_copy(ones_c, dego_sh.at[idxs_all.at[0]], ssem).wait()
        pltpu.make_async_copy(ones_c, degi_sh.at[idxd_all.at[0]], ssem).wait()
        return carry

    lax.fori_loop(0, LAG, drain, 0)
    plsc.subcore_barrier()

    pltpu.sync_copy(dego_sh.at[pl.ds(s * RPT, RPT)],
                    dego_hbm.at[c, 0, pl.ds(s * RPT, RPT)])
    pltpu.sync_copy(degi_sh.at[pl.ds(s * RPT, RPT)],
                    degi_hbm.at[c, 0, pl.ds(s * RPT, RPT)])


_deg_call = pl.kernel(
    _deg_kernel,
    out_type=[jax.ShapeDtypeStruct((NC, 1, NPAD), jnp.float32),
              jax.ShapeDtypeStruct((NC, 1, NPAD), jnp.float32)],
    mesh=_SC_MESH,
    compiler_params=_SC_PARAMS,
    scratch_types=[
        pltpu.VMEM((NITER, CHUNK), jnp.int32),
        pltpu.VMEM((NITER, CHUNK), jnp.int32),
        pltpu.VMEM((112,), jnp.float32),
        pltpu.VMEM((RPT,), jnp.float32),
        pltpu.VMEM_SHARED((NPAD,), jnp.float32),
        pltpu.VMEM_SHARED((NPAD,), jnp.float32),
        pltpu.SemaphoreType.DMA,
        pltpu.SemaphoreType.DMA,
    ],
)


def _edge_kernel(F, src_hbm, dst_hbm, tab_hbm, acc_hbm,
                 idxs_all, idxd_all, r0, r1, r2, r3, zbuf, acc_sh, tab_sh,
                 isem, g0, g1, g2, g3, s0, s1, s2, s3):
    c = lax.axis_index("c")
    s = lax.axis_index("s")
    wid = c * NS + s
    rows = [r0, r1, r2, r3]
    gsems = [g0, g1, g2, g3]
    ssems = [s0, s1, s2, s3]

    cps = pltpu.async_copy(src_hbm.at[wid], idxs_all, isem)
    cpd = pltpu.async_copy(dst_hbm.at[wid], idxd_all, isem)
    cpt = pltpu.async_copy(tab_hbm.at[pl.ds(s * RPT, RPT)],
                           tab_sh.at[pl.ds(s * RPT, RPT)], isem)

    zrows = 32
    z16 = jnp.zeros((16,), jnp.float32)

    def zero_row(i, carry):
        for jj in range(F // 16):
            zbuf[i, pl.ds(jj * 16, 16)] = z16
        return carry

    lax.fori_loop(0, zrows, zero_row, 0)

    def zero_slice(k, carry):
        pltpu.sync_copy(zbuf, acc_sh.at[pl.ds(s * RPT + k * zrows, zrows)])
        return carry

    lax.fori_loop(0, RPT // zrows, zero_slice, 0)
    cps.wait()
    cpd.wait()
    cpt.wait()
    plsc.subcore_barrier()

    _edge_ring(tab_sh, acc_sh, idxs_all, idxd_all, rows, gsems, ssems)
    plsc.subcore_barrier()

    pltpu.sync_copy(acc_sh.at[pl.ds(s * RPT, RPT)],
                    acc_hbm.at[c, pl.ds(s * RPT, RPT)])


def _make_edge_call(F):
    return pl.kernel(
        functools.partial(_edge_kernel, F),
        out_type=jax.ShapeDtypeStruct((NC, NPAD, F), jnp.float32),
        mesh=_SC_MESH,
        compiler_params=_SC_PARAMS,
        scratch_types=[
            pltpu.VMEM((NITER, CHUNK), jnp.int32),
            pltpu.VMEM((NITER, CHUNK), jnp.int32),
            pltpu.VMEM((CHUNK, F), jnp.float32),
            pltpu.VMEM((CHUNK, F), jnp.float32),
            pltpu.VMEM((CHUNK, F), jnp.float32),
            pltpu.VMEM((CHUNK, F), jnp.float32),
            pltpu.VMEM((32, F), jnp.float32),
            pltpu.VMEM_SHARED((NPAD, F), jnp.float32),
            pltpu.VMEM_SHARED((NPAD, F), jnp.float32),
        ] + [pltpu.SemaphoreType.DMA] * 9,
    )


_edge_call_h = _make_edge_call(NH)
_edge_call_c = _make_edge_call(NCLS)


def _mm_scale_body(x_ref, w_ref, norm_ref, o_ref):
    z = jnp.dot(x_ref[...], w_ref[...], preferred_element_type=jnp.float32)
    o_ref[0:NN, :] = z * norm_ref[...]
    o_ref[NN:NPAD, :] = jnp.zeros((NPAD - NN, NH), jnp.float32)


_mm_scale = pl.pallas_call(
    _mm_scale_body,
    out_shape=jax.ShapeDtypeStruct((NPAD, NH), jnp.float32),
)


def _mid_body(acc_ref, ni_ref, no_ref, b1_ref, w2_ref, o_ref):
    h = acc_ref[0] + acc_ref[1]
    h = jnp.maximum(h * ni_ref[...] + b1_ref[...], 0.0)
    o_ref[...] = jnp.dot(h * no_ref[...], w2_ref[...],
                         preferred_element_type=jnp.float32)


_mid = pl.pallas_call(
    _mid_body,
    out_shape=jax.ShapeDtypeStruct((NPAD, NCLS), jnp.float32),
)


def _fin_body(acc_ref, ni_ref, b2_ref, o_ref):
    o_ref[...] = ((acc_ref[0, :NN, :] + acc_ref[1, :NN, :])
                  * ni_ref[...] + b2_ref[...])


_fin = pl.pallas_call(
    _fin_body,
    out_shape=jax.ShapeDtypeStruct((NN, NCLS), jnp.float32),
)


@jax.jit
def kernel(features, edge_index, W1, b1, W2, b2):
    src = edge_index[0].reshape(NW, NITER, CHUNK)
    dst = edge_index[1].reshape(NW, NITER, CHUNK)

    dego, degi = _deg_call(src, dst)
    deg_out = dego[0, 0] + dego[1, 0]
    deg_in = degi[0, 0] + degi[1, 0]
    norm_out = lax.rsqrt(jnp.maximum(deg_out, 1.0)).reshape(NPAD, 1)
    norm_in = lax.rsqrt(jnp.maximum(deg_in, 1.0)).reshape(NPAD, 1)

    hw1 = _mm_scale(features, W1, norm_out[:NN])
    acc1 = _edge_call_h(src, dst, hw1)
    hw2 = _mid(acc1, norm_in, norm_out, b1.reshape(1, NH), W2)
    acc2 = _edge_call_c(src, dst, hw2)
    return _fin(acc2, norm_in[:NN], b2.reshape(1, NCLS))


# speedup vs baseline: 1.2942x; 1.0990x over previous
import functools

import jax
import jax.numpy as jnp
from jax import lax
from jax.experimental import pallas as pl
from jax.experimental.pallas import tpu as pltpu
from jax.experimental.pallas import tpu_sc as plsc

NN = 10000
NE = 320000
DF = 128
NH = 64
NCLS = 16

NC = 2
NS = 16
NW = NC * NS
EPW = NE // NW
CHUNK = 80
NITER = EPW // CHUNK
NPAD = 10240
RPT = NPAD // NS
LAG = 8

_SC_MESH = plsc.VectorSubcoreMesh(core_axis_name="c", subcore_axis_name="s")
_SC_PARAMS = pltpu.CompilerParams(use_tc_tiling_on_sc=False)


def _edge_ring(tab, acc_sh, idxs_all, idxd_all, rows, gsems, ssems):
    for b in range(3):
        pltpu.async_copy(tab.at[idxs_all.at[b]], rows[b], gsems[b])

    def body(i, carry):
        for u in range(4):
            j = 4 * i + u
            pltpu.make_async_copy(tab.at[idxs_all.at[j]], rows[u],
                                  gsems[u]).wait()
            pltpu.async_copy(rows[u], acc_sh.at[idxd_all.at[j]], ssems[u],
                             add=True)

            @pl.when(j >= 1)
            def _ws():
                pltpu.make_async_copy(rows[(u - 1) % 4],
                                      acc_sh.at[idxd_all.at[0]],
                                      ssems[(u - 1) % 4]).wait()

            @pl.when(j + 3 < NITER)
            def _g():
                pltpu.async_copy(tab.at[idxs_all.at[j + 3]],
                                 rows[(u + 3) % 4], gsems[(u + 3) % 4])

        return carry

    nfull = NITER // 4
    lax.fori_loop(0, nfull, body, 0)
    for j in range(nfull * 4, NITER):
        u = j % 4
        pltpu.make_async_copy(tab.at[idxs_all.at[j]], rows[u],
                              gsems[u]).wait()
        pltpu.async_copy(rows[u], acc_sh.at[idxd_all.at[j]], ssems[u],
                         add=True)
        pltpu.make_async_copy(rows[(u - 1) % 4], acc_sh.at[idxd_all.at[0]],
                              ssems[(u - 1) % 4]).wait()
    last = (NITER - 1) % 4
    pltpu.make_async_copy(rows[last], acc_sh.at[idxd_all.at[0]],
                          ssems[last]).wait()


def _deg_kernel(ei_hbm, dego_hbm, degi_hbm, idxs_all, idxd_all,
                ones_v, zbuf, dego_sh, degi_sh, isem, ssem):
    c = lax.axis_index("c")
    s = lax.axis_index("s")
    wid = c * NS + s

    cps = pltpu.async_copy(ei_hbm.at[0, wid], idxs_all, isem)
    cpd = pltpu.async_copy(ei_hbm.at[1, wid], idxd_all, isem)

    def fill(i, carry):
        ones_v[pl.ds(i * 16, 16)] = jnp.ones((16,), jnp.float32)
        return carry

    lax.fori_loop(0, 7, fill, 0)

    def zfill(i, carry):
        zbuf[pl.ds(i * 16, 16)] = jnp.zeros((16,), jnp.float32)
        return carry

    lax.fori_loop(0, RPT // 16, zfill, 0)
    pltpu.sync_copy(zbuf, dego_sh.at[pl.ds(s * RPT, RPT)])
    pltpu.sync_copy(zbuf, degi_sh.at[pl.ds(s * RPT, RPT)])
    cps.wait()
    cpd.wait()
    plsc.subcore_barrier()

    ones_c = ones_v.at[pl.ds(0, CHUNK)]

    def body(j, carry):
        @pl.when(j >= LAG)
        def _drain():
            pltpu.make_async_copy(ones_c, dego_sh.at[idxs_all.at[0]],
                                  ssem).wait()
            pltpu.make_async_copy(ones_c, degi_sh.at[idxd_all.at[0]],
                                  ssem).wait()

        pltpu.async_copy(ones_c, dego_sh.at[idxs_all.at[j]], ssem, add=True)
        pltpu.async_copy(ones_c, degi_sh.at[idxd_all.at[j]], ssem, add=True)
        return carry

    lax.fori_loop(0, NITER, body, 0)

    def drain(j, carry):
        pltpu.make_async_copy(ones_c, dego_sh.at[idxs_all.at[0]], ssem).wait()
        pltpu.make_async_copy(ones_c, degi_sh.at[idxd_all.at[0]], ssem).wait()
        return carry

    lax.fori_loop(0, LAG, drain, 0)
    plsc.subcore_barrier()

    pltpu.sync_copy(dego_sh.at[pl.ds(s * RPT, RPT)],
                    dego_hbm.at[c, 0, pl.ds(s * RPT, RPT)])
    pltpu.sync_copy(degi_sh.at[pl.ds(s * RPT, RPT)],
                    degi_hbm.at[c, 0, pl.ds(s * RPT, RPT)])


_deg_call = pl.kernel(
    _deg_kernel,
    out_type=[jax.ShapeDtypeStruct((NC, 1, NPAD), jnp.float32),
              jax.ShapeDtypeStruct((NC, 1, NPAD), jnp.float32)],
    mesh=_SC_MESH,
    compiler_params=_SC_PARAMS,
    scratch_types=[
        pltpu.VMEM((NITER, CHUNK), jnp.int32),
        pltpu.VMEM((NITER, CHUNK), jnp.int32),
        pltpu.VMEM((112,), jnp.float32),
        pltpu.VMEM((RPT,), jnp.float32),
        pltpu.VMEM_SHARED((NPAD,), jnp.float32),
        pltpu.VMEM_SHARED((NPAD,), jnp.float32),
        pltpu.SemaphoreType.DMA,
        pltpu.SemaphoreType.DMA,
    ],
)


def _edge_kernel(F, ei_hbm, tab_hbm, acc_hbm,
                 idxs_all, idxd_all, r0, r1, r2, r3, zbuf, acc_sh, tab_sh,
                 isem, g0, g1, g2, g3, s0, s1, s2, s3):
    c = lax.axis_index("c")
    s = lax.axis_index("s")
    wid = c * NS + s
    rows = [r0, r1, r2, r3]
    gsems = [g0, g1, g2, g3]
    ssems = [s0, s1, s2, s3]

    cps = pltpu.async_copy(ei_hbm.at[0, wid], idxs_all, isem)
    cpd = pltpu.async_copy(ei_hbm.at[1, wid], idxd_all, isem)
    cpt = pltpu.async_copy(tab_hbm.at[pl.ds(s * RPT, RPT)],
                           tab_sh.at[pl.ds(s * RPT, RPT)], isem)

    zrows = 32
    z16 = jnp.zeros((16,), jnp.float32)

    def zero_row(i, carry):
        for jj in range(F // 16):
            zbuf[i, pl.ds(jj * 16, 16)] = z16
        return carry

    lax.fori_loop(0, zrows, zero_row, 0)

    def zero_slice(k, carry):
        pltpu.sync_copy(zbuf, acc_sh.at[pl.ds(s * RPT + k * zrows, zrows)])
        return carry

    lax.fori_loop(0, RPT // zrows, zero_slice, 0)
    cps.wait()
    cpd.wait()
    cpt.wait()
    plsc.subcore_barrier()

    _edge_ring(tab_sh, acc_sh, idxs_all, idxd_all, rows, gsems, ssems)
    plsc.subcore_barrier()

    pltpu.sync_copy(acc_sh.at[pl.ds(s * RPT, RPT)],
                    acc_hbm.at[c, pl.ds(s * RPT, RPT)])


def _make_edge_call(F):
    return pl.kernel(
        functools.partial(_edge_kernel, F),
        out_type=jax.ShapeDtypeStruct((NC, NPAD, F), jnp.float32),
        mesh=_SC_MESH,
        compiler_params=_SC_PARAMS,
        scratch_types=[
            pltpu.VMEM((NITER, CHUNK), jnp.int32),
            pltpu.VMEM((NITER, CHUNK), jnp.int32),
            pltpu.VMEM((CHUNK, F), jnp.float32),
            pltpu.VMEM((CHUNK, F), jnp.float32),
            pltpu.VMEM((CHUNK, F), jnp.float32),
            pltpu.VMEM((CHUNK, F), jnp.float32),
            pltpu.VMEM((32, F), jnp.float32),
            pltpu.VMEM_SHARED((NPAD, F), jnp.float32),
            pltpu.VMEM_SHARED((NPAD, F), jnp.float32),
        ] + [pltpu.SemaphoreType.DMA] * 9,
    )


_edge_call_h = _make_edge_call(NH)
_edge_call_c = _make_edge_call(NCLS)


def _mm_scale_body(x_ref, w_ref, norm_ref, o_ref):
    z = jnp.dot(x_ref[...], w_ref[...], preferred_element_type=jnp.float32)
    o_ref[0:NN, :] = z * norm_ref[0:NN]
    o_ref[NN:NPAD, :] = jnp.zeros((NPAD - NN, NH), jnp.float32)


_mm_scale = pl.pallas_call(
    _mm_scale_body,
    out_shape=jax.ShapeDtypeStruct((NPAD, NH), jnp.float32),
)


def _mid_body(acc_ref, ni_ref, no_ref, b1_ref, w2_ref, o_ref):
    h = acc_ref[0] + acc_ref[1]
    h = jnp.maximum(h * ni_ref[...] + b1_ref[...], 0.0)
    o_ref[...] = jnp.dot(h * no_ref[...], w2_ref[...],
                         preferred_element_type=jnp.float32)


_mid = pl.pallas_call(
    _mid_body,
    out_shape=jax.ShapeDtypeStruct((NPAD, NCLS), jnp.float32),
)


def _fin_body(acc_ref, ni_ref, b2_ref, o_ref):
    o_ref[...] = ((acc_ref[0, :NN, :] + acc_ref[1, :NN, :])
                  * ni_ref[0:NN] + b2_ref[...])


_fin = pl.pallas_call(
    _fin_body,
    out_shape=jax.ShapeDtypeStruct((NN, NCLS), jnp.float32),
)


@jax.jit
def kernel(features, edge_index, W1, b1, W2, b2):
    ei = edge_index.reshape(2, NW, NITER, CHUNK)

    dego, degi = _deg_call(ei)
    deg_out = dego[0, 0] + dego[1, 0]
    deg_in = degi[0, 0] + degi[1, 0]
    norm_out = lax.rsqrt(jnp.maximum(deg_out, 1.0)).reshape(NPAD, 1)
    norm_in = lax.rsqrt(jnp.maximum(deg_in, 1.0)).reshape(NPAD, 1)

    hw1 = _mm_scale(features, W1, norm_out)
    acc1 = _edge_call_h(ei, hw1)
    hw2 = _mid(acc1, norm_in, norm_out, b1.reshape(1, NH), W2)
    acc2 = _edge_call_c(ei, hw2)
    return _fin(acc2, norm_in, b2.reshape(1, NCLS))
